# baseline (speedup 1.0000x reference)
import functools

import jax
import jax.numpy as jnp
from jax import lax
from jax.experimental import pallas as pl
from jax.experimental.pallas import tpu as pltpu
from jax.experimental.pallas import tpu_sc as plsc

NUM_EMB = 1_000_000
DIM = 32
NI = 16384
NJ = 50
NC, NS = 2, 16
NW = NC * NS
IPW = NI // NW
CHUNK = 128
NCH = IPW // CHUNK
NK = IPW // 16
PITCH = IPW + 1

_mesh = plsc.VectorSubcoreMesh(
    core_axis_name="c", subcore_axis_name="s", num_cores=NC, num_subcores=NS
)


@functools.partial(
    pl.kernel,
    out_type=jax.ShapeDtypeStruct((NJ, DIM, NI), jnp.float32),
    mesh=_mesh,
    scratch_types=[
        pltpu.VMEM((NJ, IPW), jnp.int32),
        pltpu.VMEM((2 * IPW, DIM), jnp.float32),
        pltpu.VMEM((2 * DIM, PITCH), jnp.float32),
        pltpu.SemaphoreType.DMA,
        pltpu.SemaphoreType.DMA,
    ],
    compiler_params=pltpu.CompilerParams(
        use_tc_tiling_on_sc=False, needs_layout_passes=False
    ),
)
def _emb_gather(xt_hbm, tab_hbm, out_hbm, idx_v, rows_v, tr_v, gsem, osem):
    wid = lax.axis_index("s") * NC + lax.axis_index("c")
    ib = wid * IPW
    pltpu.sync_copy(xt_hbm.at[:, pl.ds(ib, IPW)], idx_v)

    def gather_cps(j, rbase):
        return [
            pltpu.make_async_copy(
                tab_hbm.at[idx_v.at[j, pl.ds(c * CHUNK, CHUNK)]],
                rows_v.at[pl.ds(rbase + c * CHUNK, CHUNK), :],
                gsem,
            )
            for c in range(NCH)
        ]

    def out_slice(j):
        return out_hbm.at[j, :, pl.ds(ib, IPW)]

    k_iota = lax.iota(jnp.int32, 16)

    for cp in gather_cps(0, 0):
        cp.start()

    @pl.loop(0, NJ)
    def _(j):
        b = j % 2
        rbase = b * IPW
        nrbase = IPW - rbase
        tbase = b * DIM

        @pl.when(j + 1 < NJ)
        def _():
            for cp in gather_cps(j + 1, nrbase):
                cp.start()

        for cp in gather_cps(j, rbase):
            cp.wait()

        @pl.when(j >= 2)
        def _():
            pltpu.make_async_copy(
                tr_v.at[pl.ds(tbase, DIM), pl.ds(0, IPW)], out_slice(j - 2), osem
            ).wait()

        for k in range(IPW):
            col = jnp.full((16,), k, jnp.int32)
            for half in range(2):
                val = rows_v[rbase + k, pl.ds(16 * half, 16)]
                row = k_iota + (tbase + 16 * half)
                plsc.store_scatter(tr_v, [row, col], val)

        pltpu.async_copy(
            tr_v.at[pl.ds(tbase, DIM), pl.ds(0, IPW)], out_slice(j), osem
        )

    pltpu.make_async_copy(
        tr_v.at[pl.ds(0, DIM), pl.ds(0, IPW)], out_slice(NJ - 2), osem
    ).wait()
    pltpu.make_async_copy(
        tr_v.at[pl.ds(DIM, DIM), pl.ds(0, IPW)], out_slice(NJ - 1), osem
    ).wait()


TCOLS = 256
NFULL = (NUM_EMB // TCOLS) * TCOLS
NCHUNKS = NFULL // TCOLS
CPW = -(-NCHUNKS // NW)
TAIL = NUM_EMB - NFULL


@functools.partial(
    pl.kernel,
    out_type=jax.ShapeDtypeStruct((NUM_EMB * DIM,), jnp.float32),
    mesh=_mesh,
    scratch_types=[
        pltpu.VMEM((2 * DIM, TCOLS), jnp.float32),
        pltpu.VMEM((TCOLS * 33,), jnp.float32),
        pltpu.VMEM((2 * TCOLS * DIM,), jnp.float32),
        pltpu.VMEM((TAIL * DIM,), jnp.float32),
        pltpu.SemaphoreType.DMA,
        pltpu.SemaphoreType.DMA,
    ],
    compiler_params=pltpu.CompilerParams(
        use_tc_tiling_on_sc=True, needs_layout_passes=False
    ),
)
def _detile(vt_hbm, tail_hbm, out_hbm, src_v, scat_v, pack_v, tail_v, isem, osem):
    wid = lax.axis_index("s") * NC + lax.axis_index("c")
    i33 = lax.iota(jnp.int32, 16) * 33

    @pl.when(wid == 0)
    def _():
        pltpu.sync_copy(tail_hbm, tail_v)
        pltpu.sync_copy(tail_v, out_hbm.at[pl.ds(NFULL * DIM, TAIL * DIM)])

    def in_cp(t, b):
        g = t * NW + wid
        return pltpu.make_async_copy(
            vt_hbm.at[:, pl.ds(g * TCOLS, TCOLS)],
            src_v.at[pl.ds(b * DIM, DIM), :],
            isem,
        )

    def out_cp(t, b):
        g = t * NW + wid
        return pltpu.make_async_copy(
            pack_v.at[pl.ds(b * TCOLS * DIM, TCOLS * DIM)],
            out_hbm.at[pl.ds(g * TCOLS * DIM, TCOLS * DIM)],
            osem,
        )

    @pl.when(wid < NCHUNKS)
    def _():
        in_cp(0, 0).start()

    @pl.loop(0, CPW)
    def _(t):
        g = t * NW + wid
        b = t % 2

        @pl.when(g < NCHUNKS)
        def _():
            @pl.when((t + 1) * NW + wid < NCHUNKS)
            def _():
                in_cp(t + 1, 1 - b).start()

            in_cp(t, b).wait()

            @pl.when(t >= 2)
            def _():
                out_cp(t - 2, b).wait()

            rbase = b * DIM
            for d in range(DIM):
                for iv in range(TCOLS // 16):
                    val = src_v[rbase + d, pl.ds(iv * 16, 16)]
                    idx = i33 + (iv * 16 * 33 + d)
                    plsc.store_scatter(scat_v, [idx], val)
            pbase = b * TCOLS * DIM
            for i in range(TCOLS):
                for half in range(2):
                    pack_v[pl.ds(pbase + i * DIM + 16 * half, 16)] = scat_v[
                        pl.ds(i * 33 + 16 * half, 16)
                    ]
            out_cp(t, b).start()

    nch_w = (NCHUNKS - wid + NW - 1) // NW
    for delta in (2, 1):
        tl = nch_w - delta

        @pl.when(tl >= 0)
        def _():
            out_cp(tl, tl % 2).wait()


def kernel(x, vocab):
    xt = x.T.astype(jnp.int32)
    vt = vocab.T
    tail = jax.lax.slice(
        vocab, (NFULL, 0), (NUM_EMB, DIM)
    ).reshape(TAIL * DIM)
    tab = _detile(vt, tail).reshape(NUM_EMB, DIM)
    out_t = _emb_gather(xt, tab)
    return out_t.transpose(2, 0, 1)


# speedup vs baseline: 1.0448x; 1.0448x over previous
import functools

import jax
import jax.numpy as jnp
from jax import lax
from jax.experimental import pallas as pl
from jax.experimental.pallas import tpu as pltpu
from jax.experimental.pallas import tpu_sc as plsc

NUM_EMB = 1_000_000
DIM = 32
NI = 16384
NJ = 50
NC, NS = 2, 16
NW = NC * NS
IPW = NI // NW
CHUNK = 128
NCH = IPW // CHUNK
NK = IPW // 16
PITCH = IPW + 1

_mesh = plsc.VectorSubcoreMesh(
    core_axis_name="c", subcore_axis_name="s", num_cores=NC, num_subcores=NS
)


@functools.partial(
    pl.kernel,
    out_type=jax.ShapeDtypeStruct((NJ, DIM, NI), jnp.float32),
    mesh=_mesh,
    scratch_types=[
        pltpu.VMEM((NJ, IPW), jnp.int32),
        pltpu.VMEM((2 * IPW, DIM), jnp.float32),
        pltpu.VMEM((2 * DIM, PITCH), jnp.float32),
        pltpu.SemaphoreType.DMA,
        pltpu.SemaphoreType.DMA,
    ],
    compiler_params=pltpu.CompilerParams(
        use_tc_tiling_on_sc=False, needs_layout_passes=False
    ),
)
def _emb_gather(xt_hbm, tab_hbm, out_hbm, idx_v, rows_v, tr_v, gsem, osem):
    wid = lax.axis_index("s") * NC + lax.axis_index("c")
    ib = wid * IPW
    pltpu.sync_copy(xt_hbm.at[:, pl.ds(ib, IPW)], idx_v)

    def gather_cps(j, rbase):
        return [
            pltpu.make_async_copy(
                tab_hbm.at[idx_v.at[j, pl.ds(c * CHUNK, CHUNK)]],
                rows_v.at[pl.ds(rbase + c * CHUNK, CHUNK), :],
                gsem,
            )
            for c in range(NCH)
        ]

    def out_slice(j):
        return out_hbm.at[j, :, pl.ds(ib, IPW)]

    k_iota = lax.iota(jnp.int32, 16)

    for cp in gather_cps(0, 0):
        cp.start()

    @pl.loop(0, NJ)
    def _(j):
        b = j % 2
        rbase = b * IPW
        nrbase = IPW - rbase
        tbase = b * DIM

        @pl.when(j + 1 < NJ)
        def _():
            for cp in gather_cps(j + 1, nrbase):
                cp.start()

        for cp in gather_cps(j, rbase):
            cp.wait()

        @pl.when(j >= 2)
        def _():
            pltpu.make_async_copy(
                tr_v.at[pl.ds(tbase, DIM), pl.ds(0, IPW)], out_slice(j - 2), osem
            ).wait()

        for k in range(IPW):
            col = jnp.full((16,), k, jnp.int32)
            for half in range(2):
                val = rows_v[rbase + k, pl.ds(16 * half, 16)]
                row = k_iota + (tbase + 16 * half)
                plsc.store_scatter(tr_v, [row, col], val)

        pltpu.async_copy(
            tr_v.at[pl.ds(tbase, DIM), pl.ds(0, IPW)], out_slice(j), osem
        )

    pltpu.make_async_copy(
        tr_v.at[pl.ds(0, DIM), pl.ds(0, IPW)], out_slice(NJ - 2), osem
    ).wait()
    pltpu.make_async_copy(
        tr_v.at[pl.ds(DIM, DIM), pl.ds(0, IPW)], out_slice(NJ - 1), osem
    ).wait()


def kernel(x, vocab):
    xt = x.T.astype(jnp.int32)
    out_t = _emb_gather(xt, vocab)
    return out_t.transpose(2, 0, 1)
